# baseline (speedup 1.0000x reference)
import functools

import jax
import jax.numpy as jnp
from jax import lax
from jax.experimental import pallas as pl
from jax.experimental.pallas import tpu as pltpu
from jax.experimental.pallas import tpu_sc as plsc

_D = 128
_L = 16
_C = 128
_NW = 32


def _make_sc_kernel(n: int):
    n_chunks_total = -(-n // _C)
    cpw = -(-n_chunks_total // _NW)
    cpw += cpw % 2
    n_pairs = cpw // 2
    last_base = n - _C
    mesh = plsc.VectorSubcoreMesh(core_axis_name="c", subcore_axis_name="s")

    @functools.partial(
        pl.kernel,
        out_type=jax.ShapeDtypeStruct((n,), jnp.float32),
        mesh=mesh,
        compiler_params=pltpu.CompilerParams(
            needs_layout_passes=False, use_tc_tiling_on_sc=False),
        scratch_types=[
            pltpu.VMEM((2, 3 * _C), jnp.int32),
            pltpu.VMEM((2, _C), jnp.int32),
            pltpu.VMEM((2, _C), jnp.int32),
            pltpu.VMEM((2, _C), jnp.int32),
            pltpu.VMEM((2, _C, _D), jnp.bfloat16),
            pltpu.VMEM((2, _C, _D), jnp.bfloat16),
            pltpu.VMEM((2, _C, _D), jnp.bfloat16),
            pltpu.VMEM((_C,), jnp.float32),
            pltpu.SemaphoreType.DMA,
            pltpu.SemaphoreType.DMA,
            pltpu.SemaphoreType.DMA,
            pltpu.SemaphoreType.DMA,
        ],
    )
    def sc_kernel(trip_hbm, nodes_hbm, rel_hbm, out_hbm,
                  trip_v, sidx_v, pidx_v, oidx_v, s_v, p_v, o_v, out_v,
                  semt0, semt1, semr0, semr1):
        semt = (semt0, semt1)
        semr = (semr0, semr1)
        cid = lax.axis_index("c")
        sid = lax.axis_index("s")
        wid = sid * 2 + cid
        lanes = lax.iota(jnp.int32, _L)

        def chunk_base(j):
            return jnp.minimum((wid * cpw + j) * _C, last_base)

        def issue_trip(j, b):
            base = chunk_base(j) * 3
            pltpu.async_copy(trip_hbm.at[pl.ds(base, 3 * _C)], trip_v.at[b], semt[b])

        def wait_trip(b):
            pltpu.make_async_copy(trip_hbm.at[pl.ds(0, 3 * _C)], trip_v.at[b], semt[b]).wait()

        def extract_idx(b):
            src = trip_v.at[b]
            for g in range(_C // _L):
                tbase = 3 * (g * _L + lanes)
                sidx_v[b, pl.ds(g * _L, _L)] = plsc.load_gather(src, [tbase])
                pidx_v[b, pl.ds(g * _L, _L)] = plsc.load_gather(src, [tbase + 1])
                oidx_v[b, pl.ds(g * _L, _L)] = plsc.load_gather(src, [tbase + 2])

        def issue_rows(b):
            pltpu.async_copy(nodes_hbm.at[sidx_v.at[b]], s_v.at[b], semr[b])
            pltpu.async_copy(rel_hbm.at[pidx_v.at[b]], p_v.at[b], semr[b])
            pltpu.async_copy(nodes_hbm.at[oidx_v.at[b]], o_v.at[b], semr[b])

        def wait_rows(b):
            pltpu.make_async_copy(nodes_hbm.at[pl.ds(0, _C)], s_v.at[b], semr[b]).wait()
            pltpu.make_async_copy(rel_hbm.at[pl.ds(0, _C)], p_v.at[b], semr[b]).wait()
            pltpu.make_async_copy(nodes_hbm.at[pl.ds(0, _C)], o_v.at[b], semr[b]).wait()

        def compute(j, b):
            def group_body(g, carry2):
                gb = g * _L
                res = jnp.zeros((_L,), jnp.float32)
                for t in range(_L):
                    i = gb + t
                    acc = None
                    for dc in range(_D // (2 * _L)):
                        sl = pl.ds(dc * 2 * _L, 2 * _L)
                        s0, s1 = plsc.unpack(
                            s_v[b, i, sl], format=plsc.PackFormat.INTERLEAVED)
                        p0, p1 = plsc.unpack(
                            p_v[b, i, sl], format=plsc.PackFormat.INTERLEAVED)
                        o0, o1 = plsc.unpack(
                            o_v[b, i, sl], format=plsc.PackFormat.INTERLEAVED)
                        prod = s0 * p0 * o0 + s1 * p1 * o1
                        acc = prod if acc is None else acc + prod
                    res = jnp.where(lanes == t, jnp.sum(acc), res)
                out_v[pl.ds(gb, _L)] = res
                return carry2

            lax.fori_loop(0, _C // _L, group_body, 0)
            pltpu.sync_copy(out_v, out_hbm.at[pl.ds(chunk_base(j), _C)])

        issue_trip(0, 0)
        issue_trip(1, 1)
        wait_trip(0)
        extract_idx(0)
        issue_rows(0)

        def pair_body(cp, carry):
            j = cp * 2
            wait_trip(1)
            extract_idx(1)
            issue_rows(1)
            wait_rows(0)
            issue_trip(j + 2, 0)
            compute(j, 0)
            wait_trip(0)
            extract_idx(0)
            issue_rows(0)
            wait_rows(1)
            issue_trip(j + 3, 1)
            compute(j + 1, 1)
            return carry

        lax.fori_loop(0, n_pairs, pair_body, 0)
        wait_trip(1)
        wait_rows(0)

    return sc_kernel


def kernel(triples, nodes, relations):
    n = triples.shape[0]
    return _make_sc_kernel(n)(triples.reshape(-1),
                              nodes.astype(jnp.bfloat16),
                              relations.astype(jnp.bfloat16))


# speedup vs baseline: 3.7110x; 3.7110x over previous
import functools

import jax
import jax.numpy as jnp
from jax import lax
from jax.experimental import pallas as pl
from jax.experimental.pallas import tpu as pltpu
from jax.experimental.pallas import tpu_sc as plsc

_D = 128
_L = 16
_C = 128
_NW = 32


def _make_sc_kernel(n: int):
    n_chunks_total = -(-n // _C)
    cpw = -(-n_chunks_total // _NW)
    cpw += cpw % 2
    n_pairs = cpw // 2
    last_base = n - _C
    mesh = plsc.VectorSubcoreMesh(core_axis_name="c", subcore_axis_name="s")

    @functools.partial(
        pl.kernel,
        out_type=jax.ShapeDtypeStruct((n,), jnp.float32),
        mesh=mesh,
        compiler_params=pltpu.CompilerParams(
            needs_layout_passes=False, use_tc_tiling_on_sc=False),
        scratch_types=[
            pltpu.VMEM((2, _C), jnp.int32),
            pltpu.VMEM((2, _C), jnp.int32),
            pltpu.VMEM((2, _C), jnp.int32),
            pltpu.VMEM((2, _C, _D), jnp.bfloat16),
            pltpu.VMEM((2, _C, _D), jnp.bfloat16),
            pltpu.VMEM((2, _C, _D), jnp.bfloat16),
            pltpu.VMEM((_C,), jnp.float32),
            pltpu.SemaphoreType.DMA,
            pltpu.SemaphoreType.DMA,
            pltpu.SemaphoreType.DMA,
            pltpu.SemaphoreType.DMA,
        ],
    )
    def sc_kernel(sidx_hbm, pidx_hbm, oidx_hbm, nodes_hbm, rel_hbm, out_hbm,
                  sidx_v, pidx_v, oidx_v, s_v, p_v, o_v, out_v,
                  semt0, semt1, semr0, semr1):
        semt = (semt0, semt1)
        semr = (semr0, semr1)
        cid = lax.axis_index("c")
        sid = lax.axis_index("s")
        wid = sid * 2 + cid
        lanes = lax.iota(jnp.int32, _L)

        def chunk_base(j):
            return jnp.minimum((wid * cpw + j) * _C, last_base)

        def issue_trip(j, b):
            base = chunk_base(j)
            pltpu.async_copy(sidx_hbm.at[pl.ds(base, _C)], sidx_v.at[b], semt[b])
            pltpu.async_copy(pidx_hbm.at[pl.ds(base, _C)], pidx_v.at[b], semt[b])
            pltpu.async_copy(oidx_hbm.at[pl.ds(base, _C)], oidx_v.at[b], semt[b])

        def wait_trip(b):
            pltpu.make_async_copy(sidx_hbm.at[pl.ds(0, _C)], sidx_v.at[b], semt[b]).wait()
            pltpu.make_async_copy(pidx_hbm.at[pl.ds(0, _C)], pidx_v.at[b], semt[b]).wait()
            pltpu.make_async_copy(oidx_hbm.at[pl.ds(0, _C)], oidx_v.at[b], semt[b]).wait()

        def issue_rows(b):
            pltpu.async_copy(nodes_hbm.at[sidx_v.at[b]], s_v.at[b], semr[b])
            pltpu.async_copy(rel_hbm.at[pidx_v.at[b]], p_v.at[b], semr[b])
            pltpu.async_copy(nodes_hbm.at[oidx_v.at[b]], o_v.at[b], semr[b])

        def wait_rows(b):
            pltpu.make_async_copy(nodes_hbm.at[pl.ds(0, _C)], s_v.at[b], semr[b]).wait()
            pltpu.make_async_copy(rel_hbm.at[pl.ds(0, _C)], p_v.at[b], semr[b]).wait()
            pltpu.make_async_copy(nodes_hbm.at[pl.ds(0, _C)], o_v.at[b], semr[b]).wait()

        def compute(j, b):
            def group_body(g, carry2):
                gb = g * _L
                res = jnp.zeros((_L,), jnp.float32)
                for t in range(_L):
                    i = gb + t
                    acc = None
                    for dc in range(_D // (2 * _L)):
                        sl = pl.ds(dc * 2 * _L, 2 * _L)
                        s0, s1 = plsc.unpack(
                            s_v[b, i, sl], format=plsc.PackFormat.INTERLEAVED)
                        p0, p1 = plsc.unpack(
                            p_v[b, i, sl], format=plsc.PackFormat.INTERLEAVED)
                        o0, o1 = plsc.unpack(
                            o_v[b, i, sl], format=plsc.PackFormat.INTERLEAVED)
                        prod = s0 * p0 * o0 + s1 * p1 * o1
                        acc = prod if acc is None else acc + prod
                    res = jnp.where(lanes == t, jnp.sum(acc), res)
                out_v[pl.ds(gb, _L)] = res
                return carry2

            lax.fori_loop(0, _C // _L, group_body, 0)
            pltpu.sync_copy(out_v, out_hbm.at[pl.ds(chunk_base(j), _C)])

        issue_trip(0, 0)
        issue_trip(1, 1)
        wait_trip(0)
        issue_rows(0)

        def pair_body(cp, carry):
            j = cp * 2
            wait_trip(1)
            issue_rows(1)
            wait_rows(0)
            issue_trip(j + 2, 0)
            compute(j, 0)
            wait_trip(0)
            issue_rows(0)
            wait_rows(1)
            issue_trip(j + 3, 1)
            compute(j + 1, 1)
            return carry

        lax.fori_loop(0, n_pairs, pair_body, 0)
        wait_trip(1)
        wait_rows(0)

    return sc_kernel


def kernel(triples, nodes, relations):
    n = triples.shape[0]
    return _make_sc_kernel(n)(triples[:, 0], triples[:, 1], triples[:, 2],
                              nodes.astype(jnp.bfloat16),
                              relations.astype(jnp.bfloat16))
